# baseline (speedup 1.0000x reference)
import functools

import jax
import jax.numpy as jnp
from jax import lax
from jax.experimental import pallas as pl
from jax.experimental.pallas import tpu as pltpu
from jax.experimental.pallas import tpu_sc as plsc

N = 10000
E = 320000
B = 50
NPG = 200
H = 128
K = 100

NC, NS, L = 2, 16, 16
NW = NC * NS
CHUNK = 128
PBUF = 4
NCHUNK = 80
EPW = NCHUNK * CHUNK
EP = EPW * NW
NP = 10240
RPS = NP // NS
CH0 = 160


def _mesh():
    return plsc.VectorSubcoreMesh(
        core_axis_name="c", subcore_axis_name="s", num_cores=NC, num_subcores=NS)


def _sc_apply_feat(table, packed3):
    @functools.partial(
        pl.kernel,
        out_type=jax.ShapeDtypeStruct((NP, H), jnp.float32),
        mesh=_mesh(),
        scratch_types=(
            [pltpu.VMEM((3, CHUNK), jnp.int32) for _ in range(PBUF)]
            + [pltpu.VMEM((CHUNK, H), jnp.float32) for _ in range(2)]
            + [pltpu.VMEM_SHARED((NP, H), jnp.float32)]
            + [pltpu.SemaphoreType.DMA for _ in range(PBUF + 5)]
        ),
        compiler_params=pltpu.CompilerParams(needs_layout_passes=False),
    )
    def body(table_h, packed3_h, out_h, pb0, pb1, pb2, pb3,
             rows0, rows1, acc_sh,
             p0, p1, p2, p3, ga, gb, sa, sb, wsem):
        pbuf = (pb0, pb1, pb2, pb3)
        psem = (p0, p1, p2, p3)
        rowsb = (rows0, rows1)
        gsem = (ga, gb)
        ssem = (sa, sb)
        cid = lax.axis_index("c")
        sid = lax.axis_index("s")
        nch = CH0
        cbase = sid * CH0

        @pl.when(cid == 0)
        def _sc0_only():
            with jax.named_scope("zeroinit"):
                def _zrow(e, carry):
                    for j in range(H // L):
                        rows0[e, pl.ds(j * L, L)] = jnp.zeros((L,), jnp.float32)
                    return carry
                lax.fori_loop(0, CHUNK, _zrow, None)
                r0 = sid * RPS
                for q in range(RPS // CHUNK):
                    pltpu.sync_copy(rows0, acc_sh.at[pl.ds(r0 + q * CHUNK, CHUNK)])
                plsc.subcore_barrier()

            def pstart(c, slot):
                pltpu.make_async_copy(
                    packed3_h.at[cbase + c], pbuf[slot], psem[slot]).start()

            def pwait(c, slot):
                pltpu.make_async_copy(
                    packed3_h.at[cbase + c], pbuf[slot], psem[slot]).wait()

            def gstart(slot, rb):
                pltpu.make_async_copy(
                    table_h.at[pbuf[slot].at[0]], rowsb[rb], gsem[rb]).start()

            def gwait(slot, rb):
                pltpu.make_async_copy(
                    table_h.at[pbuf[slot].at[0]], rowsb[rb], gsem[rb]).wait()

            def sstart(slot, rb):
                pltpu.make_async_copy(
                    rowsb[rb], acc_sh.at[pbuf[slot].at[1]], ssem[rb]).start(add=True)

            def swait(slot, rb):
                pltpu.make_async_copy(
                    rowsb[rb], acc_sh.at[pbuf[slot].at[1]], ssem[rb]).wait()

            for c in range(3):
                pstart(c, c)
            pwait(0, 0)
            gstart(0, 0)

            def outer(o, carry):
                for b0 in range(PBUF):
                    c = o * PBUF + b0
                    rb = b0 % 2
                    gwait(b0, rb)
                    if b0 == 0:
                        @pl.when(c >= 1)
                        def _():
                            swait((b0 + 3) % PBUF, 1 - rb)
                    else:
                        swait((b0 + 3) % PBUF, 1 - rb)

                    @pl.when(c + 1 < nch)
                    def _():
                        pwait(c + 1, (b0 + 1) % PBUF)
                        gstart((b0 + 1) % PBUF, 1 - rb)

                    @plsc.parallel_loop(0, CHUNK, unroll=4)
                    def _medge(e):
                        vsp = plsc.bitcast(plsc.load_gather(
                            pbuf[b0],
                            [jnp.broadcast_to(2, (L,)).astype(jnp.int32),
                             jnp.broadcast_to(e, (L,)).astype(jnp.int32)]),
                            jnp.float32)
                        rv = rowsb[rb]
                        for j in range(H // L):
                            sl = pl.ds(j * L, L)
                            rv[e, sl] = rv[e, sl] * vsp

                    sstart(b0, rb)

                    @pl.when(c + 3 < nch)
                    def _():
                        pstart(c + 3, (b0 + 3) % PBUF)
                return carry
            with jax.named_scope("mainloop"):
                lax.fori_loop(0, nch // PBUF, outer, None)
                swait(PBUF - 1, 1)

            with jax.named_scope("writeout"):
                plsc.subcore_barrier()
                for q in range(RPS // CHUNK):
                    sl = pl.ds(r0 + q * CHUNK, CHUNK)
                    pltpu.make_async_copy(
                        acc_sh.at[sl], out_h.at[sl], wsem).start()
                for q in range(RPS // CHUNK):
                    sl = pl.ds(r0 + q * CHUNK, CHUNK)
                    pltpu.make_async_copy(
                        acc_sh.at[sl], out_h.at[sl], wsem).wait()

    return body(table, packed3)


def _sc_apply_scalar(table1, packed3):
    @functools.partial(
        pl.kernel,
        out_type=jax.ShapeDtypeStruct((NC, NP), jnp.float32),
        mesh=_mesh(),
        scratch_types=(
            [pltpu.VMEM((3, CHUNK), jnp.int32) for _ in range(PBUF)]
            + [pltpu.VMEM((CHUNK,), jnp.float32) for _ in range(2)]
            + [pltpu.VMEM_SHARED((NP,), jnp.float32)]
            + [pltpu.SemaphoreType.DMA for _ in range(PBUF + 4)]
        ),
        compiler_params=pltpu.CompilerParams(needs_layout_passes=False),
    )
    def body(table_h, packed3_h, out_h, pb0, pb1, pb2, pb3,
             msg0, msg1, acc_sh, p0, p1, p2, p3, ga, gb, sa, sb):
        pbuf = (pb0, pb1, pb2, pb3)
        psem = (p0, p1, p2, p3)
        msgb = (msg0, msg1)
        gsem = (ga, gb)
        ssem = (sa, sb)
        cid = lax.axis_index("c")
        sid = lax.axis_index("s")
        wid = cid * NS + sid
        cbase = wid * NCHUNK

        for g in range(CHUNK // L):
            msg0[pl.ds(g * L, L)] = jnp.zeros((L,), jnp.float32)
        r0 = sid * RPS
        for q in range(RPS // CHUNK):
            pltpu.sync_copy(msg0, acc_sh.at[pl.ds(r0 + q * CHUNK, CHUNK)])
        plsc.subcore_barrier()

        def pstart(c, slot):
            pltpu.make_async_copy(
                packed3_h.at[cbase + c], pbuf[slot], psem[slot]).start()

        def pwait(c, slot):
            pltpu.make_async_copy(
                packed3_h.at[cbase + c], pbuf[slot], psem[slot]).wait()

        def gstart(slot, rb):
            pltpu.make_async_copy(
                table_h.at[pbuf[slot].at[0]], msgb[rb], gsem[rb]).start()

        def gwait(slot, rb):
            pltpu.make_async_copy(
                table_h.at[pbuf[slot].at[0]], msgb[rb], gsem[rb]).wait()

        def sstart(slot, rb):
            pltpu.make_async_copy(
                msgb[rb], acc_sh.at[pbuf[slot].at[1]], ssem[rb]).start(add=True)

        def swait(slot, rb):
            pltpu.make_async_copy(
                msgb[rb], acc_sh.at[pbuf[slot].at[1]], ssem[rb]).wait()

        for c in range(3):
            pstart(c, c)
        pwait(0, 0)
        gstart(0, 0)

        def outer(o, carry):
            for b0 in range(PBUF):
                c = o * PBUF + b0
                rb = b0 % 2
                gwait(b0, rb)
                if b0 == 0:
                    @pl.when(c >= 1)
                    def _():
                        swait((b0 + 3) % PBUF, 1 - rb)
                else:
                    swait((b0 + 3) % PBUF, 1 - rb)

                @pl.when(c + 1 < NCHUNK)
                def _():
                    pwait(c + 1, (b0 + 1) % PBUF)
                    gstart((b0 + 1) % PBUF, 1 - rb)

                mv = msgb[rb]
                for g in range(CHUNK // L):
                    sl = pl.ds(g * L, L)
                    mv[sl] = mv[sl] * plsc.bitcast(pbuf[b0][2, sl], jnp.float32)
                sstart(b0, rb)

                @pl.when(c + 3 < NCHUNK)
                def _():
                    pstart(c + 3, (b0 + 3) % PBUF)
            return carry
        lax.fori_loop(0, NCHUNK // PBUF, outer, None)
        swait((NCHUNK - 1) % PBUF, (NCHUNK - 1) % 2)

        plsc.subcore_barrier()
        for q in range(RPS // CHUNK):
            sl = pl.ds(r0 + q * CHUNK, CHUNK)
            pltpu.sync_copy(acc_sh.at[sl], out_h.at[cid, sl])

    return body(table1, packed3)


BM = 1000


def _tc_matmul(x, w):
    def body(x_ref, w_ref, o_ref):
        o_ref[...] = jnp.dot(x_ref[...], w_ref[...],
                             preferred_element_type=jnp.float32)
    return pl.pallas_call(
        body,
        grid=(N // BM,),
        in_specs=[pl.BlockSpec((BM, H), lambda i: (i, 0)),
                  pl.BlockSpec((H, H), lambda i: (0, 0))],
        out_specs=pl.BlockSpec((BM, H), lambda i: (i, 0)),
        out_shape=jax.ShapeDtypeStruct((N, H), jnp.float32),
    )(x, w)


def _tc_epilogue_matmul(p0, b, w):
    def body(p0_ref, b_ref, w_ref, g_ref, s_ref):
        g = jnp.maximum(p0_ref[...] + b_ref[...], 0.0)
        g_ref[...] = g
        s_ref[...] = jnp.dot(g, w_ref[...], preferred_element_type=jnp.float32)
    return pl.pallas_call(
        body,
        grid=(N // BM,),
        in_specs=[pl.BlockSpec((BM, H), lambda i: (i, 0)),
                  pl.BlockSpec((1, H), lambda i: (0, 0)),
                  pl.BlockSpec((H, H), lambda i: (0, 0))],
        out_specs=[pl.BlockSpec((BM, H), lambda i: (i, 0)),
                   pl.BlockSpec((BM, H), lambda i: (i, 0))],
        out_shape=[jax.ShapeDtypeStruct((N, H), jnp.float32),
                   jax.ShapeDtypeStruct((N, H), jnp.float32)],
    )(p0, b, w)


def _tc_attn_prep(p0, b3, g1, g2, wa):
    def body(p0_ref, b_ref, g1_ref, g2_ref, wa_ref, g3_ref, aw_ref):
        g3 = jnp.maximum(p0_ref[...] + b_ref[...], 0.0)
        g3_ref[...] = g3
        wa = wa_ref[...]
        aw = (jnp.sum(g1_ref[...] * wa[0:1, :], axis=1, keepdims=True)
              + jnp.sum(g2_ref[...] * wa[1:2, :], axis=1, keepdims=True)
              + jnp.sum(g3 * wa[2:3, :], axis=1, keepdims=True))
        aw_ref[...] = aw
    return pl.pallas_call(
        body,
        grid=(N // BM,),
        in_specs=[pl.BlockSpec((BM, H), lambda i: (i, 0)),
                  pl.BlockSpec((1, H), lambda i: (0, 0)),
                  pl.BlockSpec((BM, H), lambda i: (i, 0)),
                  pl.BlockSpec((BM, H), lambda i: (i, 0)),
                  pl.BlockSpec((3, H), lambda i: (0, 0))],
        out_specs=[pl.BlockSpec((BM, H), lambda i: (i, 0)),
                   pl.BlockSpec((BM, 1), lambda i: (i, 0))],
        out_shape=[jax.ShapeDtypeStruct((N, H), jnp.float32),
                   jax.ShapeDtypeStruct((N, 1), jnp.float32)],
    )(p0, b3, g1, g2, wa)


def _tc_readout(attn_row, attn_col, g1, g2, g3):
    def body(ar_ref, ac_ref, g1_ref, g2_ref, g3_ref, out_ref, imp_ref):
        a_row = jnp.reshape(ar_ref[...], (1, NPG))
        a_col = jnp.reshape(ac_ref[...], (NPG, 1))
        i_row = lax.broadcasted_iota(jnp.int32, (NPG, NPG), 1)
        i_col = lax.broadcasted_iota(jnp.int32, (NPG, NPG), 0)
        gt = (a_row > a_col).astype(jnp.float32)
        tie = ((a_row == a_col) & (i_row < i_col)).astype(jnp.float32)
        rank = jnp.sum(gt + tie, axis=1, keepdims=True)

        r_iota = lax.broadcasted_iota(jnp.int32, (NPG, K), 1).astype(jnp.float32)
        n_iota = lax.broadcasted_iota(jnp.int32, (NPG, K), 0).astype(jnp.float32)
        onehot = (rank == r_iota).astype(jnp.float32)
        imp = jnp.sum(onehot * n_iota, axis=0, keepdims=True)
        imp_ref[...] = jnp.reshape(imp.astype(jnp.int32), (1, 1, K))

        keep = rank < float(K)
        keepf = keep.astype(jnp.float32)
        outs = []
        for g_ref in (g1_ref, g2_ref, g3_ref):
            hid = g_ref[...] * a_col
            outs.append(jnp.sum(hid * keepf, axis=0, keepdims=True) / float(K))
        for g_ref in (g1_ref, g2_ref, g3_ref):
            hid = g_ref[...] * a_col
            masked = jnp.where(keep, hid, -jnp.inf)
            outs.append(jnp.max(masked, axis=0, keepdims=True))
        out_ref[...] = jnp.reshape(jnp.concatenate(outs, axis=1), (1, 1, 6 * H))

    return pl.pallas_call(
        body,
        grid=(B,),
        in_specs=[pl.BlockSpec((1, 1, NPG), lambda i: (i, 0, 0)),
                  pl.BlockSpec((NPG, 1), lambda i: (i, 0)),
                  pl.BlockSpec((NPG, H), lambda i: (i, 0)),
                  pl.BlockSpec((NPG, H), lambda i: (i, 0)),
                  pl.BlockSpec((NPG, H), lambda i: (i, 0))],
        out_specs=[pl.BlockSpec((1, 1, 6 * H), lambda i: (i, 0, 0)),
                   pl.BlockSpec((1, 1, K), lambda i: (i, 0, 0))],
        out_shape=[jax.ShapeDtypeStruct((B, 1, 6 * H), jnp.float32),
                   jax.ShapeDtypeStruct((B, 1, K), jnp.int32)],
    )(attn_row, attn_col, g1, g2, g3)


def kernel(input_feature, adj_indices, adj_values, graph_indicator,
           W1, b1, W2, b2, W3, b3, Wa, ba):
    row = adj_indices[0].astype(jnp.int32)
    col = adj_indices[1].astype(jnp.int32)
    val = adj_values.astype(jnp.float32)
    pad = EP - E
    colp = jnp.concatenate([col, jnp.zeros((pad,), jnp.int32)])
    rowp = jnp.concatenate([row, jnp.zeros((pad,), jnp.int32)])
    valp = jnp.concatenate([val, jnp.zeros((pad,), jnp.float32)])
    packed3 = jnp.stack(
        [colp.reshape(NW * NCHUNK, CHUNK),
         rowp.reshape(NW * NCHUNK, CHUNK),
         lax.bitcast_convert_type(valp, jnp.int32).reshape(NW * NCHUNK, CHUNK)],
        axis=1)

    b1r = b1.reshape(1, H)
    b2r = b2.reshape(1, H)
    b3r = b3.reshape(1, H)
    waT = Wa.reshape(3, H)

    s1 = _tc_matmul(input_feature, W1)
    p1 = _sc_apply_feat(s1, packed3)
    g1, s2 = _tc_epilogue_matmul(p1[:N], b1r, W2)
    p2 = _sc_apply_feat(s2, packed3)
    g2, s3 = _tc_epilogue_matmul(p2[:N], b2r, W3)
    p3 = _sc_apply_feat(s3, packed3)
    g3, aw = _tc_attn_prep(p3[:N], b3r, g1, g2, waT)

    pa = _sc_apply_scalar(aw.reshape(N), packed3)
    pre = pa[0, :N] + pa[1, :N] + ba[0]
    attn = jnp.tanh(pre)

    readout3, imp3 = _tc_readout(attn.reshape(B, 1, NPG), attn.reshape(N, 1),
                                 g1, g2, g3)
    return readout3.reshape(B, 6 * H), imp3.reshape(B * K)


# speedup vs baseline: 1.0044x; 1.0044x over previous
import functools

import jax
import jax.numpy as jnp
from jax import lax
from jax.experimental import pallas as pl
from jax.experimental.pallas import tpu as pltpu
from jax.experimental.pallas import tpu_sc as plsc

N = 10000
E = 320000
B = 50
NPG = 200
H = 128
K = 100

NC, NS, L = 2, 16, 16
NW = NC * NS
CHUNK = 128
PBUF = 4
NCHUNK = 80
EPW = NCHUNK * CHUNK
EP = EPW * NW
NP = 10240
RPS = NP // NS
CH0 = 160


def _mesh():
    return plsc.VectorSubcoreMesh(
        core_axis_name="c", subcore_axis_name="s", num_cores=NC, num_subcores=NS)


def _sc_apply_feat(table, packed3):
    @functools.partial(
        pl.kernel,
        out_type=jax.ShapeDtypeStruct((NC, NP, H), jnp.float32),
        mesh=_mesh(),
        scratch_types=(
            [pltpu.VMEM((3, CHUNK), jnp.int32) for _ in range(PBUF)]
            + [pltpu.VMEM((CHUNK, H), jnp.float32) for _ in range(2)]
            + [pltpu.VMEM_SHARED((NP, H), jnp.float32)]
            + [pltpu.SemaphoreType.DMA for _ in range(PBUF + 5)]
        ),
        compiler_params=pltpu.CompilerParams(needs_layout_passes=False),
    )
    def body(table_h, packed3_h, out_h, pb0, pb1, pb2, pb3,
             rows0, rows1, acc_sh,
             p0, p1, p2, p3, ga, gb, sa, sb, wsem):
        pbuf = (pb0, pb1, pb2, pb3)
        psem = (p0, p1, p2, p3)
        rowsb = (rows0, rows1)
        gsem = (ga, gb)
        ssem = (sa, sb)
        cid = lax.axis_index("c")
        sid = lax.axis_index("s")
        nch = CH0
        cbase = sid * CH0

        @pl.when(cid == 0)
        def _sc0_only():
            with jax.named_scope("zeroinit"):
                def _zrow(e, carry):
                    for j in range(H // L):
                        rows0[e, pl.ds(j * L, L)] = jnp.zeros((L,), jnp.float32)
                    return carry
                lax.fori_loop(0, CHUNK, _zrow, None)
                r0 = sid * RPS
                for q in range(RPS // CHUNK):
                    pltpu.sync_copy(rows0, acc_sh.at[pl.ds(r0 + q * CHUNK, CHUNK)])
                plsc.subcore_barrier()

            def pstart(c, slot):
                pltpu.make_async_copy(
                    packed3_h.at[cbase + c], pbuf[slot], psem[slot]).start()

            def pwait(c, slot):
                pltpu.make_async_copy(
                    packed3_h.at[cbase + c], pbuf[slot], psem[slot]).wait()

            def gstart(slot, rb):
                pltpu.make_async_copy(
                    table_h.at[pbuf[slot].at[0]], rowsb[rb], gsem[rb]).start()

            def gwait(slot, rb):
                pltpu.make_async_copy(
                    table_h.at[pbuf[slot].at[0]], rowsb[rb], gsem[rb]).wait()

            def sstart(slot, rb):
                pltpu.make_async_copy(
                    rowsb[rb], acc_sh.at[pbuf[slot].at[1]], ssem[rb]).start(add=True)

            def swait(slot, rb):
                pltpu.make_async_copy(
                    rowsb[rb], acc_sh.at[pbuf[slot].at[1]], ssem[rb]).wait()

            for c in range(3):
                pstart(c, c)
            pwait(0, 0)
            gstart(0, 0)

            def outer(o, carry):
                for b0 in range(PBUF):
                    c = o * PBUF + b0
                    rb = b0 % 2
                    gwait(b0, rb)
                    if b0 == 0:
                        @pl.when(c >= 1)
                        def _():
                            swait((b0 + 3) % PBUF, 1 - rb)
                    else:
                        swait((b0 + 3) % PBUF, 1 - rb)

                    @pl.when(c + 1 < nch)
                    def _():
                        pwait(c + 1, (b0 + 1) % PBUF)
                        gstart((b0 + 1) % PBUF, 1 - rb)

                    @plsc.parallel_loop(0, CHUNK, unroll=4)
                    def _medge(e):
                        vsp = plsc.bitcast(plsc.load_gather(
                            pbuf[b0],
                            [jnp.broadcast_to(2, (L,)).astype(jnp.int32),
                             jnp.broadcast_to(e, (L,)).astype(jnp.int32)]),
                            jnp.float32)
                        rv = rowsb[rb]
                        for j in range(H // L):
                            sl = pl.ds(j * L, L)
                            rv[e, sl] = rv[e, sl] * vsp

                    sstart(b0, rb)

                    @pl.when(c + 3 < nch)
                    def _():
                        pstart(c + 3, (b0 + 3) % PBUF)
                return carry
            with jax.named_scope("mainloop"):
                lax.fori_loop(0, nch // PBUF, outer, None)
                swait(PBUF - 1, 1)

            with jax.named_scope("writeout"):
                plsc.subcore_barrier()
                for q in range(RPS // CHUNK):
                    sl = pl.ds(r0 + q * CHUNK, CHUNK)
                    pltpu.make_async_copy(
                        acc_sh.at[sl], out_h.at[0, sl], wsem).start()
                for q in range(RPS // CHUNK):
                    sl = pl.ds(r0 + q * CHUNK, CHUNK)
                    pltpu.make_async_copy(
                        acc_sh.at[sl], out_h.at[0, sl], wsem).wait()

    return body(table, packed3)


def _sc_apply_scalar(table1, packed3):
    @functools.partial(
        pl.kernel,
        out_type=jax.ShapeDtypeStruct((NC, NP), jnp.float32),
        mesh=_mesh(),
        scratch_types=(
            [pltpu.VMEM((3, CHUNK), jnp.int32) for _ in range(PBUF)]
            + [pltpu.VMEM((CHUNK,), jnp.float32) for _ in range(2)]
            + [pltpu.VMEM_SHARED((NP,), jnp.float32)]
            + [pltpu.SemaphoreType.DMA for _ in range(PBUF + 4)]
        ),
        compiler_params=pltpu.CompilerParams(needs_layout_passes=False),
    )
    def body(table_h, packed3_h, out_h, pb0, pb1, pb2, pb3,
             msg0, msg1, acc_sh, p0, p1, p2, p3, ga, gb, sa, sb):
        pbuf = (pb0, pb1, pb2, pb3)
        psem = (p0, p1, p2, p3)
        msgb = (msg0, msg1)
        gsem = (ga, gb)
        ssem = (sa, sb)
        cid = lax.axis_index("c")
        sid = lax.axis_index("s")
        wid = cid * NS + sid
        cbase = wid * NCHUNK

        for g in range(CHUNK // L):
            msg0[pl.ds(g * L, L)] = jnp.zeros((L,), jnp.float32)
        r0 = sid * RPS
        for q in range(RPS // CHUNK):
            pltpu.sync_copy(msg0, acc_sh.at[pl.ds(r0 + q * CHUNK, CHUNK)])
        plsc.subcore_barrier()

        def pstart(c, slot):
            pltpu.make_async_copy(
                packed3_h.at[cbase + c], pbuf[slot], psem[slot]).start()

        def pwait(c, slot):
            pltpu.make_async_copy(
                packed3_h.at[cbase + c], pbuf[slot], psem[slot]).wait()

        def gstart(slot, rb):
            pltpu.make_async_copy(
                table_h.at[pbuf[slot].at[0]], msgb[rb], gsem[rb]).start()

        def gwait(slot, rb):
            pltpu.make_async_copy(
                table_h.at[pbuf[slot].at[0]], msgb[rb], gsem[rb]).wait()

        def sstart(slot, rb):
            pltpu.make_async_copy(
                msgb[rb], acc_sh.at[pbuf[slot].at[1]], ssem[rb]).start(add=True)

        def swait(slot, rb):
            pltpu.make_async_copy(
                msgb[rb], acc_sh.at[pbuf[slot].at[1]], ssem[rb]).wait()

        for c in range(3):
            pstart(c, c)
        pwait(0, 0)
        gstart(0, 0)

        def outer(o, carry):
            for b0 in range(PBUF):
                c = o * PBUF + b0
                rb = b0 % 2
                gwait(b0, rb)
                if b0 == 0:
                    @pl.when(c >= 1)
                    def _():
                        swait((b0 + 3) % PBUF, 1 - rb)
                else:
                    swait((b0 + 3) % PBUF, 1 - rb)

                @pl.when(c + 1 < NCHUNK)
                def _():
                    pwait(c + 1, (b0 + 1) % PBUF)
                    gstart((b0 + 1) % PBUF, 1 - rb)

                mv = msgb[rb]
                for g in range(CHUNK // L):
                    sl = pl.ds(g * L, L)
                    mv[sl] = mv[sl] * plsc.bitcast(pbuf[b0][2, sl], jnp.float32)
                sstart(b0, rb)

                @pl.when(c + 3 < NCHUNK)
                def _():
                    pstart(c + 3, (b0 + 3) % PBUF)
            return carry
        lax.fori_loop(0, NCHUNK // PBUF, outer, None)
        swait((NCHUNK - 1) % PBUF, (NCHUNK - 1) % 2)

        plsc.subcore_barrier()
        for q in range(RPS // CHUNK):
            sl = pl.ds(r0 + q * CHUNK, CHUNK)
            pltpu.sync_copy(acc_sh.at[sl], out_h.at[cid, sl])

    return body(table1, packed3)


BM = 1000


def _tc_matmul(x, w):
    def body(x_ref, w_ref, o_ref):
        o_ref[...] = jnp.dot(x_ref[...], w_ref[...],
                             preferred_element_type=jnp.float32)
    return pl.pallas_call(
        body,
        grid=(N // BM,),
        in_specs=[pl.BlockSpec((BM, H), lambda i: (i, 0)),
                  pl.BlockSpec((H, H), lambda i: (0, 0))],
        out_specs=pl.BlockSpec((BM, H), lambda i: (i, 0)),
        out_shape=jax.ShapeDtypeStruct((N, H), jnp.float32),
    )(x, w)


def _tc_epilogue_matmul(p0, b, w):
    def body(p0_ref, b_ref, w_ref, g_ref, s_ref):
        g = jnp.maximum(p0_ref[...] + b_ref[...], 0.0)
        g_ref[...] = g
        s_ref[...] = jnp.dot(g, w_ref[...], preferred_element_type=jnp.float32)
    return pl.pallas_call(
        body,
        grid=(N // BM,),
        in_specs=[pl.BlockSpec((BM, H), lambda i: (i, 0)),
                  pl.BlockSpec((1, H), lambda i: (0, 0)),
                  pl.BlockSpec((H, H), lambda i: (0, 0))],
        out_specs=[pl.BlockSpec((BM, H), lambda i: (i, 0)),
                   pl.BlockSpec((BM, H), lambda i: (i, 0))],
        out_shape=[jax.ShapeDtypeStruct((N, H), jnp.float32),
                   jax.ShapeDtypeStruct((N, H), jnp.float32)],
    )(p0, b, w)


def _tc_attn_prep(p0, b3, g1, g2, wa):
    def body(p0_ref, b_ref, g1_ref, g2_ref, wa_ref, g3_ref, aw_ref):
        g3 = jnp.maximum(p0_ref[...] + b_ref[...], 0.0)
        g3_ref[...] = g3
        wa = wa_ref[...]
        aw = (jnp.sum(g1_ref[...] * wa[0:1, :], axis=1, keepdims=True)
              + jnp.sum(g2_ref[...] * wa[1:2, :], axis=1, keepdims=True)
              + jnp.sum(g3 * wa[2:3, :], axis=1, keepdims=True))
        aw_ref[...] = aw
    return pl.pallas_call(
        body,
        grid=(N // BM,),
        in_specs=[pl.BlockSpec((BM, H), lambda i: (i, 0)),
                  pl.BlockSpec((1, H), lambda i: (0, 0)),
                  pl.BlockSpec((BM, H), lambda i: (i, 0)),
                  pl.BlockSpec((BM, H), lambda i: (i, 0)),
                  pl.BlockSpec((3, H), lambda i: (0, 0))],
        out_specs=[pl.BlockSpec((BM, H), lambda i: (i, 0)),
                   pl.BlockSpec((BM, 1), lambda i: (i, 0))],
        out_shape=[jax.ShapeDtypeStruct((N, H), jnp.float32),
                   jax.ShapeDtypeStruct((N, 1), jnp.float32)],
    )(p0, b3, g1, g2, wa)


def _tc_readout(attn_row, attn_col, g1, g2, g3):
    def body(ar_ref, ac_ref, g1_ref, g2_ref, g3_ref, out_ref, imp_ref):
        a_row = jnp.reshape(ar_ref[...], (1, NPG))
        a_col = jnp.reshape(ac_ref[...], (NPG, 1))
        i_row = lax.broadcasted_iota(jnp.int32, (NPG, NPG), 1)
        i_col = lax.broadcasted_iota(jnp.int32, (NPG, NPG), 0)
        gt = (a_row > a_col).astype(jnp.float32)
        tie = ((a_row == a_col) & (i_row < i_col)).astype(jnp.float32)
        rank = jnp.sum(gt + tie, axis=1, keepdims=True)

        r_iota = lax.broadcasted_iota(jnp.int32, (NPG, K), 1).astype(jnp.float32)
        n_iota = lax.broadcasted_iota(jnp.int32, (NPG, K), 0).astype(jnp.float32)
        onehot = (rank == r_iota).astype(jnp.float32)
        imp = jnp.sum(onehot * n_iota, axis=0, keepdims=True)
        imp_ref[...] = jnp.reshape(imp.astype(jnp.int32), (1, 1, K))

        keep = rank < float(K)
        keepf = keep.astype(jnp.float32)
        outs = []
        for g_ref in (g1_ref, g2_ref, g3_ref):
            hid = g_ref[...] * a_col
            outs.append(jnp.sum(hid * keepf, axis=0, keepdims=True) / float(K))
        for g_ref in (g1_ref, g2_ref, g3_ref):
            hid = g_ref[...] * a_col
            masked = jnp.where(keep, hid, -jnp.inf)
            outs.append(jnp.max(masked, axis=0, keepdims=True))
        out_ref[...] = jnp.reshape(jnp.concatenate(outs, axis=1), (1, 1, 6 * H))

    return pl.pallas_call(
        body,
        grid=(B,),
        in_specs=[pl.BlockSpec((1, 1, NPG), lambda i: (i, 0, 0)),
                  pl.BlockSpec((NPG, 1), lambda i: (i, 0)),
                  pl.BlockSpec((NPG, H), lambda i: (i, 0)),
                  pl.BlockSpec((NPG, H), lambda i: (i, 0)),
                  pl.BlockSpec((NPG, H), lambda i: (i, 0))],
        out_specs=[pl.BlockSpec((1, 1, 6 * H), lambda i: (i, 0, 0)),
                   pl.BlockSpec((1, 1, K), lambda i: (i, 0, 0))],
        out_shape=[jax.ShapeDtypeStruct((B, 1, 6 * H), jnp.float32),
                   jax.ShapeDtypeStruct((B, 1, K), jnp.int32)],
    )(attn_row, attn_col, g1, g2, g3)


def kernel(input_feature, adj_indices, adj_values, graph_indicator,
           W1, b1, W2, b2, W3, b3, Wa, ba):
    row = adj_indices[0].astype(jnp.int32)
    col = adj_indices[1].astype(jnp.int32)
    val = adj_values.astype(jnp.float32)
    pad = EP - E
    colp = jnp.concatenate([col, jnp.zeros((pad,), jnp.int32)])
    rowp = jnp.concatenate([row, jnp.zeros((pad,), jnp.int32)])
    valp = jnp.concatenate([val, jnp.zeros((pad,), jnp.float32)])
    packed3 = jnp.stack(
        [colp.reshape(NW * NCHUNK, CHUNK),
         rowp.reshape(NW * NCHUNK, CHUNK),
         lax.bitcast_convert_type(valp, jnp.int32).reshape(NW * NCHUNK, CHUNK)],
        axis=1)

    b1r = b1.reshape(1, H)
    b2r = b2.reshape(1, H)
    b3r = b3.reshape(1, H)
    waT = Wa.reshape(3, H)

    s1 = _tc_matmul(input_feature, W1)
    p1 = _sc_apply_feat(s1, packed3)
    g1, s2 = _tc_epilogue_matmul(p1[0, :N], b1r, W2)
    p2 = _sc_apply_feat(s2, packed3)
    g2, s3 = _tc_epilogue_matmul(p2[0, :N], b2r, W3)
    p3 = _sc_apply_feat(s3, packed3)
    g3, aw = _tc_attn_prep(p3[0, :N], b3r, g1, g2, waT)

    pa = _sc_apply_scalar(aw.reshape(N), packed3)
    pre = pa[0, :N] + pa[1, :N] + ba[0]
    attn = jnp.tanh(pre)

    readout3, imp3 = _tc_readout(attn.reshape(B, 1, NPG), attn.reshape(N, 1),
                                 g1, g2, g3)
    return readout3.reshape(B, 6 * H), imp3.reshape(B * K)


# speedup vs baseline: 1.1982x; 1.1929x over previous
import functools

import jax
import jax.numpy as jnp
from jax import lax
from jax.experimental import pallas as pl
from jax.experimental.pallas import tpu as pltpu
from jax.experimental.pallas import tpu_sc as plsc

N = 10000
E = 320000
B = 50
NPG = 200
H = 128
K = 100

NC, NS, L = 2, 16, 16
NW = NC * NS
CHUNK = 128
PBUF = 4
NCHUNK = 80
EPW = NCHUNK * CHUNK
EP = EPW * NW
NP = 10240
RPS = NP // NS
CH0, CH1 = 112, 48


def _mesh():
    return plsc.VectorSubcoreMesh(
        core_axis_name="c", subcore_axis_name="s", num_cores=NC, num_subcores=NS)


def _sc_apply_feat(table, packed3):
    @functools.partial(
        pl.kernel,
        out_type=jax.ShapeDtypeStruct((NC, NP, H), jnp.float32),
        mesh=_mesh(),
        scratch_types=(
            [pltpu.VMEM((3, CHUNK), jnp.int32) for _ in range(PBUF)]
            + [pltpu.VMEM((CHUNK, H), jnp.float32) for _ in range(2)]
            + [pltpu.VMEM_SHARED((NP, H), jnp.float32)]
            + [pltpu.SemaphoreType.DMA for _ in range(PBUF + 4)]
        ),
        compiler_params=pltpu.CompilerParams(needs_layout_passes=False),
    )
    def body(table_h, packed3_h, out_h, pb0, pb1, pb2, pb3,
             rows0, rows1, acc_sh,
             p0, p1, p2, p3, ga, gb, sa, sb):
        pbuf = (pb0, pb1, pb2, pb3)
        psem = (p0, p1, p2, p3)
        rowsb = (rows0, rows1)
        gsem = (ga, gb)
        ssem = (sa, sb)
        cid = lax.axis_index("c")
        sid = lax.axis_index("s")
        nch = jnp.where(cid == 0, CH0, CH1)
        cbase = jnp.where(cid == 0, sid * CH0, NS * CH0 + sid * CH1)

        def _zrow(e, carry):
            for j in range(H // L):
                rows0[e, pl.ds(j * L, L)] = jnp.zeros((L,), jnp.float32)
            return carry
        lax.fori_loop(0, CHUNK, _zrow, None)
        r0 = sid * RPS
        for q in range(RPS // CHUNK):
            pltpu.sync_copy(rows0, acc_sh.at[pl.ds(r0 + q * CHUNK, CHUNK)])
        plsc.subcore_barrier()

        def pstart(c, slot):
            pltpu.make_async_copy(
                packed3_h.at[cbase + c], pbuf[slot], psem[slot]).start()

        def pwait(c, slot):
            pltpu.make_async_copy(
                packed3_h.at[cbase + c], pbuf[slot], psem[slot]).wait()

        def gstart(slot, rb):
            pltpu.make_async_copy(
                table_h.at[pbuf[slot].at[0]], rowsb[rb], gsem[rb]).start()

        def gwait(slot, rb):
            pltpu.make_async_copy(
                table_h.at[pbuf[slot].at[0]], rowsb[rb], gsem[rb]).wait()

        def sstart(slot, rb):
            pltpu.make_async_copy(
                rowsb[rb], acc_sh.at[pbuf[slot].at[1]], ssem[rb]).start(add=True)

        def swait(slot, rb):
            pltpu.make_async_copy(
                rowsb[rb], acc_sh.at[pbuf[slot].at[1]], ssem[rb]).wait()

        for c in range(3):
            pstart(c, c)
        pwait(0, 0)
        gstart(0, 0)

        def outer(o, carry):
            for b0 in range(PBUF):
                c = o * PBUF + b0
                rb = b0 % 2
                gwait(b0, rb)
                if b0 == 0:
                    @pl.when(c >= 1)
                    def _():
                        swait((b0 + 3) % PBUF, 1 - rb)
                else:
                    swait((b0 + 3) % PBUF, 1 - rb)

                @pl.when(c + 1 < nch)
                def _():
                    pwait(c + 1, (b0 + 1) % PBUF)
                    gstart((b0 + 1) % PBUF, 1 - rb)

                @plsc.parallel_loop(0, CHUNK, unroll=4)
                def _medge(e):
                    vsp = plsc.bitcast(plsc.load_gather(
                        pbuf[b0],
                        [jnp.broadcast_to(2, (L,)).astype(jnp.int32),
                         jnp.broadcast_to(e, (L,)).astype(jnp.int32)]),
                        jnp.float32)
                    rv = rowsb[rb]
                    for j in range(H // L):
                        sl = pl.ds(j * L, L)
                        rv[e, sl] = rv[e, sl] * vsp

                sstart(b0, rb)

                @pl.when(c + 3 < nch)
                def _():
                    pstart(c + 3, (b0 + 3) % PBUF)
            return carry
        lax.fori_loop(0, nch // PBUF, outer, None)
        swait(PBUF - 1, 1)

        plsc.subcore_barrier()
        for q in range(RPS // CHUNK):
            sl = pl.ds(r0 + q * CHUNK, CHUNK)
            pltpu.sync_copy(acc_sh.at[sl], out_h.at[cid, sl])

    return body(table, packed3)


def _sc_apply_scalar(table1, packed3):
    @functools.partial(
        pl.kernel,
        out_type=jax.ShapeDtypeStruct((NC, NP), jnp.float32),
        mesh=_mesh(),
        scratch_types=(
            [pltpu.VMEM((3, CHUNK), jnp.int32) for _ in range(PBUF)]
            + [pltpu.VMEM((CHUNK,), jnp.float32) for _ in range(2)]
            + [pltpu.VMEM_SHARED((NP,), jnp.float32)]
            + [pltpu.SemaphoreType.DMA for _ in range(PBUF + 4)]
        ),
        compiler_params=pltpu.CompilerParams(needs_layout_passes=False),
    )
    def body(table_h, packed3_h, out_h, pb0, pb1, pb2, pb3,
             msg0, msg1, acc_sh, p0, p1, p2, p3, ga, gb, sa, sb):
        pbuf = (pb0, pb1, pb2, pb3)
        psem = (p0, p1, p2, p3)
        msgb = (msg0, msg1)
        gsem = (ga, gb)
        ssem = (sa, sb)
        cid = lax.axis_index("c")
        sid = lax.axis_index("s")
        wid = cid * NS + sid
        cbase = wid * NCHUNK

        for g in range(CHUNK // L):
            msg0[pl.ds(g * L, L)] = jnp.zeros((L,), jnp.float32)
        r0 = sid * RPS
        for q in range(RPS // CHUNK):
            pltpu.sync_copy(msg0, acc_sh.at[pl.ds(r0 + q * CHUNK, CHUNK)])
        plsc.subcore_barrier()

        def pstart(c, slot):
            pltpu.make_async_copy(
                packed3_h.at[cbase + c], pbuf[slot], psem[slot]).start()

        def pwait(c, slot):
            pltpu.make_async_copy(
                packed3_h.at[cbase + c], pbuf[slot], psem[slot]).wait()

        def gstart(slot, rb):
            pltpu.make_async_copy(
                table_h.at[pbuf[slot].at[0]], msgb[rb], gsem[rb]).start()

        def gwait(slot, rb):
            pltpu.make_async_copy(
                table_h.at[pbuf[slot].at[0]], msgb[rb], gsem[rb]).wait()

        def sstart(slot, rb):
            pltpu.make_async_copy(
                msgb[rb], acc_sh.at[pbuf[slot].at[1]], ssem[rb]).start(add=True)

        def swait(slot, rb):
            pltpu.make_async_copy(
                msgb[rb], acc_sh.at[pbuf[slot].at[1]], ssem[rb]).wait()

        for c in range(3):
            pstart(c, c)
        pwait(0, 0)
        gstart(0, 0)

        def outer(o, carry):
            for b0 in range(PBUF):
                c = o * PBUF + b0
                rb = b0 % 2
                gwait(b0, rb)
                if b0 == 0:
                    @pl.when(c >= 1)
                    def _():
                        swait((b0 + 3) % PBUF, 1 - rb)
                else:
                    swait((b0 + 3) % PBUF, 1 - rb)

                @pl.when(c + 1 < NCHUNK)
                def _():
                    pwait(c + 1, (b0 + 1) % PBUF)
                    gstart((b0 + 1) % PBUF, 1 - rb)

                mv = msgb[rb]
                for g in range(CHUNK // L):
                    sl = pl.ds(g * L, L)
                    mv[sl] = mv[sl] * plsc.bitcast(pbuf[b0][2, sl], jnp.float32)
                sstart(b0, rb)

                @pl.when(c + 3 < NCHUNK)
                def _():
                    pstart(c + 3, (b0 + 3) % PBUF)
            return carry
        lax.fori_loop(0, NCHUNK // PBUF, outer, None)
        swait((NCHUNK - 1) % PBUF, (NCHUNK - 1) % 2)

        plsc.subcore_barrier()
        for q in range(RPS // CHUNK):
            sl = pl.ds(r0 + q * CHUNK, CHUNK)
            pltpu.sync_copy(acc_sh.at[sl], out_h.at[cid, sl])

    return body(table1, packed3)


BM = 1000


def _tc_matmul(x, w):
    def body(x_ref, w_ref, o_ref):
        o_ref[...] = jnp.dot(x_ref[...], w_ref[...],
                             preferred_element_type=jnp.float32)
    return pl.pallas_call(
        body,
        grid=(N // BM,),
        in_specs=[pl.BlockSpec((BM, H), lambda i: (i, 0)),
                  pl.BlockSpec((H, H), lambda i: (0, 0))],
        out_specs=pl.BlockSpec((BM, H), lambda i: (i, 0)),
        out_shape=jax.ShapeDtypeStruct((N, H), jnp.float32),
    )(x, w)


def _tc_epilogue_matmul(p0, p1, b, w):
    def body(p0_ref, p1_ref, b_ref, w_ref, g_ref, s_ref):
        g = jnp.maximum(p0_ref[...] + p1_ref[...] + b_ref[...], 0.0)
        g_ref[...] = g
        s_ref[...] = jnp.dot(g, w_ref[...], preferred_element_type=jnp.float32)
    return pl.pallas_call(
        body,
        grid=(N // BM,),
        in_specs=[pl.BlockSpec((BM, H), lambda i: (i, 0)),
                  pl.BlockSpec((BM, H), lambda i: (i, 0)),
                  pl.BlockSpec((1, H), lambda i: (0, 0)),
                  pl.BlockSpec((H, H), lambda i: (0, 0))],
        out_specs=[pl.BlockSpec((BM, H), lambda i: (i, 0)),
                   pl.BlockSpec((BM, H), lambda i: (i, 0))],
        out_shape=[jax.ShapeDtypeStruct((N, H), jnp.float32),
                   jax.ShapeDtypeStruct((N, H), jnp.float32)],
    )(p0, p1, b, w)


def _tc_attn_prep(p0, p1, b3, g1, g2, wa):
    def body(p0_ref, p1_ref, b_ref, g1_ref, g2_ref, wa_ref, g3_ref, aw_ref):
        g3 = jnp.maximum(p0_ref[...] + p1_ref[...] + b_ref[...], 0.0)
        g3_ref[...] = g3
        wa = wa_ref[...]
        aw = (jnp.sum(g1_ref[...] * wa[0:1, :], axis=1, keepdims=True)
              + jnp.sum(g2_ref[...] * wa[1:2, :], axis=1, keepdims=True)
              + jnp.sum(g3 * wa[2:3, :], axis=1, keepdims=True))
        aw_ref[...] = aw
    return pl.pallas_call(
        body,
        grid=(N // BM,),
        in_specs=[pl.BlockSpec((BM, H), lambda i: (i, 0)),
                  pl.BlockSpec((BM, H), lambda i: (i, 0)),
                  pl.BlockSpec((1, H), lambda i: (0, 0)),
                  pl.BlockSpec((BM, H), lambda i: (i, 0)),
                  pl.BlockSpec((BM, H), lambda i: (i, 0)),
                  pl.BlockSpec((3, H), lambda i: (0, 0))],
        out_specs=[pl.BlockSpec((BM, H), lambda i: (i, 0)),
                   pl.BlockSpec((BM, 1), lambda i: (i, 0))],
        out_shape=[jax.ShapeDtypeStruct((N, H), jnp.float32),
                   jax.ShapeDtypeStruct((N, 1), jnp.float32)],
    )(p0, p1, b3, g1, g2, wa)


def _tc_readout(attn_row, attn_col, g1, g2, g3):
    def body(ar_ref, ac_ref, g1_ref, g2_ref, g3_ref, out_ref, imp_ref):
        a_row = jnp.reshape(ar_ref[...], (1, NPG))
        a_col = jnp.reshape(ac_ref[...], (NPG, 1))
        i_row = lax.broadcasted_iota(jnp.int32, (NPG, NPG), 1)
        i_col = lax.broadcasted_iota(jnp.int32, (NPG, NPG), 0)
        gt = (a_row > a_col).astype(jnp.float32)
        tie = ((a_row == a_col) & (i_row < i_col)).astype(jnp.float32)
        rank = jnp.sum(gt + tie, axis=1, keepdims=True)

        r_iota = lax.broadcasted_iota(jnp.int32, (NPG, K), 1).astype(jnp.float32)
        n_iota = lax.broadcasted_iota(jnp.int32, (NPG, K), 0).astype(jnp.float32)
        onehot = (rank == r_iota).astype(jnp.float32)
        imp = jnp.sum(onehot * n_iota, axis=0, keepdims=True)
        imp_ref[...] = jnp.reshape(imp.astype(jnp.int32), (1, 1, K))

        keep = rank < float(K)
        keepf = keep.astype(jnp.float32)
        outs = []
        for g_ref in (g1_ref, g2_ref, g3_ref):
            hid = g_ref[...] * a_col
            outs.append(jnp.sum(hid * keepf, axis=0, keepdims=True) / float(K))
        for g_ref in (g1_ref, g2_ref, g3_ref):
            hid = g_ref[...] * a_col
            masked = jnp.where(keep, hid, -jnp.inf)
            outs.append(jnp.max(masked, axis=0, keepdims=True))
        out_ref[...] = jnp.reshape(jnp.concatenate(outs, axis=1), (1, 1, 6 * H))

    return pl.pallas_call(
        body,
        grid=(B,),
        in_specs=[pl.BlockSpec((1, 1, NPG), lambda i: (i, 0, 0)),
                  pl.BlockSpec((NPG, 1), lambda i: (i, 0)),
                  pl.BlockSpec((NPG, H), lambda i: (i, 0)),
                  pl.BlockSpec((NPG, H), lambda i: (i, 0)),
                  pl.BlockSpec((NPG, H), lambda i: (i, 0))],
        out_specs=[pl.BlockSpec((1, 1, 6 * H), lambda i: (i, 0, 0)),
                   pl.BlockSpec((1, 1, K), lambda i: (i, 0, 0))],
        out_shape=[jax.ShapeDtypeStruct((B, 1, 6 * H), jnp.float32),
                   jax.ShapeDtypeStruct((B, 1, K), jnp.int32)],
    )(attn_row, attn_col, g1, g2, g3)


def kernel(input_feature, adj_indices, adj_values, graph_indicator,
           W1, b1, W2, b2, W3, b3, Wa, ba):
    row = adj_indices[0].astype(jnp.int32)
    col = adj_indices[1].astype(jnp.int32)
    val = adj_values.astype(jnp.float32)
    pad = EP - E
    colp = jnp.concatenate([col, jnp.zeros((pad,), jnp.int32)])
    rowp = jnp.concatenate([row, jnp.zeros((pad,), jnp.int32)])
    valp = jnp.concatenate([val, jnp.zeros((pad,), jnp.float32)])
    packed3 = jnp.stack(
        [colp.reshape(NW * NCHUNK, CHUNK),
         rowp.reshape(NW * NCHUNK, CHUNK),
         lax.bitcast_convert_type(valp, jnp.int32).reshape(NW * NCHUNK, CHUNK)],
        axis=1)

    b1r = b1.reshape(1, H)
    b2r = b2.reshape(1, H)
    b3r = b3.reshape(1, H)
    waT = Wa.reshape(3, H)

    s1 = _tc_matmul(input_feature, W1)
    p1 = _sc_apply_feat(s1, packed3)
    g1, s2 = _tc_epilogue_matmul(p1[0, :N], p1[1, :N], b1r, W2)
    p2 = _sc_apply_feat(s2, packed3)
    g2, s3 = _tc_epilogue_matmul(p2[0, :N], p2[1, :N], b2r, W3)
    p3 = _sc_apply_feat(s3, packed3)
    g3, aw = _tc_attn_prep(p3[0, :N], p3[1, :N], b3r, g1, g2, waT)

    pa = _sc_apply_scalar(aw.reshape(N), packed3)
    pre = pa[0, :N] + pa[1, :N] + ba[0]
    attn = jnp.tanh(pre)

    readout3, imp3 = _tc_readout(attn.reshape(B, 1, NPG), attn.reshape(N, 1),
                                 g1, g2, g3)
    return readout3.reshape(B, 6 * H), imp3.reshape(B * K)


# speedup vs baseline: 1.2430x; 1.0375x over previous
import functools

import jax
import jax.numpy as jnp
from jax import lax
from jax.experimental import pallas as pl
from jax.experimental.pallas import tpu as pltpu
from jax.experimental.pallas import tpu_sc as plsc

N = 10000
E = 320000
B = 50
NPG = 200
H = 128
K = 100

NC, NS, L = 2, 16, 16
NW = NC * NS
CHUNK = 128
PBUF = 4
NCHUNK = 80
EPW = NCHUNK * CHUNK
EP = EPW * NW
NP = 10240
RPS = NP // NS
CH0, CH1 = 112, 48


def _mesh():
    return plsc.VectorSubcoreMesh(
        core_axis_name="c", subcore_axis_name="s", num_cores=NC, num_subcores=NS)


def _sc_apply_feat(table, packed3):
    @functools.partial(
        pl.kernel,
        out_type=[jax.ShapeDtypeStruct((NP, H), jnp.float32),
                  jax.ShapeDtypeStruct((NP, H // 2), jnp.int32)],
        mesh=_mesh(),
        scratch_types=(
            [pltpu.VMEM((3, CHUNK), jnp.int32) for _ in range(PBUF)]
            + [pltpu.VMEM((CHUNK, H), jnp.float32) for _ in range(2)]
            + [pltpu.VMEM((CHUNK // 2, H // 2), jnp.int32)]
            + [pltpu.VMEM_SHARED((NP, H), jnp.float32)]
            + [pltpu.SemaphoreType.DMA for _ in range(PBUF + 4)]
        ),
        compiler_params=pltpu.CompilerParams(needs_layout_passes=False),
    )
    def body(table_h, packed3_h, outf_h, outb_h, pb0, pb1, pb2, pb3,
             rows0, rows1, pkbuf, acc_sh,
             p0, p1, p2, p3, ga, gb, sa, sb):
        pbuf = (pb0, pb1, pb2, pb3)
        psem = (p0, p1, p2, p3)
        rowsb = (rows0, rows1)
        gsem = (ga, gb)
        ssem = (sa, sb)
        cid = lax.axis_index("c")
        sid = lax.axis_index("s")
        nch = jnp.where(cid == 0, CH0, CH1)
        cbase = jnp.where(cid == 0, sid * CH0, NS * CH0 + sid * CH1)

        def _zrow(e, carry):
            for j in range(H // L):
                rows0[e, pl.ds(j * L, L)] = jnp.zeros((L,), jnp.float32)
            return carry
        lax.fori_loop(0, CHUNK, _zrow, None)
        r0 = sid * RPS
        for q in range(RPS // CHUNK):
            pltpu.sync_copy(rows0, acc_sh.at[pl.ds(r0 + q * CHUNK, CHUNK)])
        plsc.subcore_barrier()

        def pstart(c, slot):
            pltpu.make_async_copy(
                packed3_h.at[cbase + c], pbuf[slot], psem[slot]).start()

        def pwait(c, slot):
            pltpu.make_async_copy(
                packed3_h.at[cbase + c], pbuf[slot], psem[slot]).wait()

        def gstart(slot, rb):
            pltpu.make_async_copy(
                table_h.at[pbuf[slot].at[0]], rowsb[rb], gsem[rb]).start()

        def gwait(slot, rb):
            pltpu.make_async_copy(
                table_h.at[pbuf[slot].at[0]], rowsb[rb], gsem[rb]).wait()

        def sstart(slot, rb):
            pltpu.make_async_copy(
                rowsb[rb], acc_sh.at[pbuf[slot].at[1]], ssem[rb]).start(add=True)

        def swait(slot, rb):
            pltpu.make_async_copy(
                rowsb[rb], acc_sh.at[pbuf[slot].at[1]], ssem[rb]).wait()

        for c in range(3):
            pstart(c, c)
        pwait(0, 0)
        gstart(0, 0)

        def outer(o, carry):
            for b0 in range(PBUF):
                c = o * PBUF + b0
                rb = b0 % 2
                gwait(b0, rb)
                if b0 == 0:
                    @pl.when(c >= 1)
                    def _():
                        swait((b0 + 3) % PBUF, 1 - rb)
                else:
                    swait((b0 + 3) % PBUF, 1 - rb)

                @pl.when(c + 1 < nch)
                def _():
                    pwait(c + 1, (b0 + 1) % PBUF)
                    gstart((b0 + 1) % PBUF, 1 - rb)

                @plsc.parallel_loop(0, CHUNK, unroll=4)
                def _medge(e):
                    vsp = plsc.bitcast(plsc.load_gather(
                        pbuf[b0],
                        [jnp.broadcast_to(2, (L,)).astype(jnp.int32),
                         jnp.broadcast_to(e, (L,)).astype(jnp.int32)]),
                        jnp.float32)
                    rv = rowsb[rb]
                    for j in range(H // L):
                        sl = pl.ds(j * L, L)
                        rv[e, sl] = rv[e, sl] * vsp

                sstart(b0, rb)

                @pl.when(c + 3 < nch)
                def _():
                    pstart(c + 3, (b0 + 3) % PBUF)
            return carry
        lax.fori_loop(0, nch // PBUF, outer, None)
        swait(PBUF - 1, 1)

        plsc.subcore_barrier()

        @pl.when(cid == 0)
        def _w0():
            for q in range(RPS // CHUNK):
                sl = pl.ds(r0 + q * CHUNK, CHUNK)
                pltpu.sync_copy(acc_sh.at[sl], outf_h.at[sl])

        @pl.when(cid == 1)
        def _w1():
            for q in range(RPS // CHUNK):
                sl = pl.ds(r0 + q * CHUNK, CHUNK)
                pltpu.sync_copy(acc_sh.at[sl], rows0)
                for h in range(2):
                    def _prow(r, carry):
                        for j in range(H // 32):
                            a = plsc.bitcast(
                                rows0[h * (CHUNK // 2) + r, pl.ds(32 * j, L)],
                                jnp.int32)
                            b = plsc.bitcast(
                                rows0[h * (CHUNK // 2) + r,
                                      pl.ds(32 * j + L, L)], jnp.int32)
                            ap = lax.shift_right_logical(a + 0x8000, 16)
                            bp = lax.bitwise_and(b + 0x8000, jnp.int32(-65536))
                            pkbuf[r, pl.ds(L * j, L)] = lax.bitwise_or(ap, bp)
                        return carry
                    lax.fori_loop(0, CHUNK // 2, _prow, None)
                    slh = pl.ds(r0 + q * CHUNK + h * (CHUNK // 2), CHUNK // 2)
                    pltpu.sync_copy(pkbuf, outb_h.at[slh])

    return body(table, packed3)


def _sc_apply_scalar(table1, packed3):
    @functools.partial(
        pl.kernel,
        out_type=jax.ShapeDtypeStruct((NC, NP), jnp.float32),
        mesh=_mesh(),
        scratch_types=(
            [pltpu.VMEM((3, CHUNK), jnp.int32) for _ in range(PBUF)]
            + [pltpu.VMEM((CHUNK,), jnp.float32) for _ in range(2)]
            + [pltpu.VMEM_SHARED((NP,), jnp.float32)]
            + [pltpu.SemaphoreType.DMA for _ in range(PBUF + 4)]
        ),
        compiler_params=pltpu.CompilerParams(needs_layout_passes=False),
    )
    def body(table_h, packed3_h, out_h, pb0, pb1, pb2, pb3,
             msg0, msg1, acc_sh, p0, p1, p2, p3, ga, gb, sa, sb):
        pbuf = (pb0, pb1, pb2, pb3)
        psem = (p0, p1, p2, p3)
        msgb = (msg0, msg1)
        gsem = (ga, gb)
        ssem = (sa, sb)
        cid = lax.axis_index("c")
        sid = lax.axis_index("s")
        wid = cid * NS + sid
        cbase = wid * NCHUNK

        for g in range(CHUNK // L):
            msg0[pl.ds(g * L, L)] = jnp.zeros((L,), jnp.float32)
        r0 = sid * RPS
        for q in range(RPS // CHUNK):
            pltpu.sync_copy(msg0, acc_sh.at[pl.ds(r0 + q * CHUNK, CHUNK)])
        plsc.subcore_barrier()

        def pstart(c, slot):
            pltpu.make_async_copy(
                packed3_h.at[cbase + c], pbuf[slot], psem[slot]).start()

        def pwait(c, slot):
            pltpu.make_async_copy(
                packed3_h.at[cbase + c], pbuf[slot], psem[slot]).wait()

        def gstart(slot, rb):
            pltpu.make_async_copy(
                table_h.at[pbuf[slot].at[0]], msgb[rb], gsem[rb]).start()

        def gwait(slot, rb):
            pltpu.make_async_copy(
                table_h.at[pbuf[slot].at[0]], msgb[rb], gsem[rb]).wait()

        def sstart(slot, rb):
            pltpu.make_async_copy(
                msgb[rb], acc_sh.at[pbuf[slot].at[1]], ssem[rb]).start(add=True)

        def swait(slot, rb):
            pltpu.make_async_copy(
                msgb[rb], acc_sh.at[pbuf[slot].at[1]], ssem[rb]).wait()

        for c in range(3):
            pstart(c, c)
        pwait(0, 0)
        gstart(0, 0)

        def outer(o, carry):
            for b0 in range(PBUF):
                c = o * PBUF + b0
                rb = b0 % 2
                gwait(b0, rb)
                if b0 == 0:
                    @pl.when(c >= 1)
                    def _():
                        swait((b0 + 3) % PBUF, 1 - rb)
                else:
                    swait((b0 + 3) % PBUF, 1 - rb)

                @pl.when(c + 1 < NCHUNK)
                def _():
                    pwait(c + 1, (b0 + 1) % PBUF)
                    gstart((b0 + 1) % PBUF, 1 - rb)

                mv = msgb[rb]
                for g in range(CHUNK // L):
                    sl = pl.ds(g * L, L)
                    mv[sl] = mv[sl] * plsc.bitcast(pbuf[b0][2, sl], jnp.float32)
                sstart(b0, rb)

                @pl.when(c + 3 < NCHUNK)
                def _():
                    pstart(c + 3, (b0 + 3) % PBUF)
            return carry
        lax.fori_loop(0, NCHUNK // PBUF, outer, None)
        swait((NCHUNK - 1) % PBUF, (NCHUNK - 1) % 2)

        plsc.subcore_barrier()
        for q in range(RPS // CHUNK):
            sl = pl.ds(r0 + q * CHUNK, CHUNK)
            pltpu.sync_copy(acc_sh.at[sl], out_h.at[cid, sl])

    return body(table1, packed3)


BM = 1000


def _tc_matmul(x, w):
    def body(x_ref, w_ref, o_ref):
        o_ref[...] = jnp.dot(x_ref[...], w_ref[...],
                             preferred_element_type=jnp.float32)
    return pl.pallas_call(
        body,
        grid=(N // BM,),
        in_specs=[pl.BlockSpec((BM, H), lambda i: (i, 0)),
                  pl.BlockSpec((H, H), lambda i: (0, 0))],
        out_specs=pl.BlockSpec((BM, H), lambda i: (i, 0)),
        out_shape=jax.ShapeDtypeStruct((N, H), jnp.float32),
    )(x, w)


def _tc_epilogue_matmul(p0, p1, b, w):
    def body(p0_ref, p1_ref, b_ref, w_ref, g_ref, s_ref):
        g = jnp.maximum(p0_ref[...] + p1_ref[...] + b_ref[...], 0.0)
        g_ref[...] = g
        s_ref[...] = jnp.dot(g, w_ref[...], preferred_element_type=jnp.float32)
    return pl.pallas_call(
        body,
        grid=(N // BM,),
        in_specs=[pl.BlockSpec((BM, H), lambda i: (i, 0)),
                  pl.BlockSpec((BM, H), lambda i: (i, 0)),
                  pl.BlockSpec((1, H), lambda i: (0, 0)),
                  pl.BlockSpec((H, H), lambda i: (0, 0))],
        out_specs=[pl.BlockSpec((BM, H), lambda i: (i, 0)),
                   pl.BlockSpec((BM, H), lambda i: (i, 0))],
        out_shape=[jax.ShapeDtypeStruct((N, H), jnp.float32),
                   jax.ShapeDtypeStruct((N, H), jnp.float32)],
    )(p0, p1, b, w)


def _tc_attn_prep(p0, p1, b3, g1, g2, wa):
    def body(p0_ref, p1_ref, b_ref, g1_ref, g2_ref, wa_ref, g3_ref, aw_ref):
        g3 = jnp.maximum(p0_ref[...] + p1_ref[...] + b_ref[...], 0.0)
        g3_ref[...] = g3
        wa = wa_ref[...]
        aw = (jnp.sum(g1_ref[...] * wa[0:1, :], axis=1, keepdims=True)
              + jnp.sum(g2_ref[...] * wa[1:2, :], axis=1, keepdims=True)
              + jnp.sum(g3 * wa[2:3, :], axis=1, keepdims=True))
        aw_ref[...] = aw
    return pl.pallas_call(
        body,
        grid=(N // BM,),
        in_specs=[pl.BlockSpec((BM, H), lambda i: (i, 0)),
                  pl.BlockSpec((BM, H), lambda i: (i, 0)),
                  pl.BlockSpec((1, H), lambda i: (0, 0)),
                  pl.BlockSpec((BM, H), lambda i: (i, 0)),
                  pl.BlockSpec((BM, H), lambda i: (i, 0)),
                  pl.BlockSpec((3, H), lambda i: (0, 0))],
        out_specs=[pl.BlockSpec((BM, H), lambda i: (i, 0)),
                   pl.BlockSpec((BM, 1), lambda i: (i, 0))],
        out_shape=[jax.ShapeDtypeStruct((N, H), jnp.float32),
                   jax.ShapeDtypeStruct((N, 1), jnp.float32)],
    )(p0, p1, b3, g1, g2, wa)


def _tc_readout(attn_row, attn_col, g1, g2, g3):
    def body(ar_ref, ac_ref, g1_ref, g2_ref, g3_ref, out_ref, imp_ref):
        a_row = jnp.reshape(ar_ref[...], (1, NPG))
        a_col = jnp.reshape(ac_ref[...], (NPG, 1))
        i_row = lax.broadcasted_iota(jnp.int32, (NPG, NPG), 1)
        i_col = lax.broadcasted_iota(jnp.int32, (NPG, NPG), 0)
        gt = (a_row > a_col).astype(jnp.float32)
        tie = ((a_row == a_col) & (i_row < i_col)).astype(jnp.float32)
        rank = jnp.sum(gt + tie, axis=1, keepdims=True)

        r_iota = lax.broadcasted_iota(jnp.int32, (NPG, K), 1).astype(jnp.float32)
        n_iota = lax.broadcasted_iota(jnp.int32, (NPG, K), 0).astype(jnp.float32)
        onehot = (rank == r_iota).astype(jnp.float32)
        imp = jnp.sum(onehot * n_iota, axis=0, keepdims=True)
        imp_ref[...] = jnp.reshape(imp.astype(jnp.int32), (1, 1, K))

        keep = rank < float(K)
        keepf = keep.astype(jnp.float32)
        outs = []
        for g_ref in (g1_ref, g2_ref, g3_ref):
            hid = g_ref[...] * a_col
            outs.append(jnp.sum(hid * keepf, axis=0, keepdims=True) / float(K))
        for g_ref in (g1_ref, g2_ref, g3_ref):
            hid = g_ref[...] * a_col
            masked = jnp.where(keep, hid, -jnp.inf)
            outs.append(jnp.max(masked, axis=0, keepdims=True))
        out_ref[...] = jnp.reshape(jnp.concatenate(outs, axis=1), (1, 1, 6 * H))

    return pl.pallas_call(
        body,
        grid=(B,),
        in_specs=[pl.BlockSpec((1, 1, NPG), lambda i: (i, 0, 0)),
                  pl.BlockSpec((NPG, 1), lambda i: (i, 0)),
                  pl.BlockSpec((NPG, H), lambda i: (i, 0)),
                  pl.BlockSpec((NPG, H), lambda i: (i, 0)),
                  pl.BlockSpec((NPG, H), lambda i: (i, 0))],
        out_specs=[pl.BlockSpec((1, 1, 6 * H), lambda i: (i, 0, 0)),
                   pl.BlockSpec((1, 1, K), lambda i: (i, 0, 0))],
        out_shape=[jax.ShapeDtypeStruct((B, 1, 6 * H), jnp.float32),
                   jax.ShapeDtypeStruct((B, 1, K), jnp.int32)],
    )(attn_row, attn_col, g1, g2, g3)


def kernel(input_feature, adj_indices, adj_values, graph_indicator,
           W1, b1, W2, b2, W3, b3, Wa, ba):
    row = adj_indices[0].astype(jnp.int32)
    col = adj_indices[1].astype(jnp.int32)
    val = adj_values.astype(jnp.float32)
    pad = EP - E
    colp = jnp.concatenate([col, jnp.zeros((pad,), jnp.int32)])
    rowp = jnp.concatenate([row, jnp.zeros((pad,), jnp.int32)])
    valp = jnp.concatenate([val, jnp.zeros((pad,), jnp.float32)])
    packed3 = jnp.stack(
        [colp.reshape(NW * NCHUNK, CHUNK),
         rowp.reshape(NW * NCHUNK, CHUNK),
         lax.bitcast_convert_type(valp, jnp.int32).reshape(NW * NCHUNK, CHUNK)],
        axis=1)

    b1r = b1.reshape(1, H)
    b2r = b2.reshape(1, H)
    b3r = b3.reshape(1, H)
    waT = Wa.reshape(3, H)

    def _decode_half(pi):
        pb = lax.bitcast_convert_type(pi[:N], jnp.bfloat16)
        return pb.reshape(N, H // 32, 16, 2).transpose(0, 1, 3, 2) \
                 .reshape(N, H).astype(jnp.float32)

    s1 = _tc_matmul(input_feature, W1)
    p1f, p1b = _sc_apply_feat(s1, packed3)
    g1, s2 = _tc_epilogue_matmul(p1f[:N], _decode_half(p1b), b1r, W2)
    p2f, p2b = _sc_apply_feat(s2, packed3)
    g2, s3 = _tc_epilogue_matmul(p2f[:N], _decode_half(p2b), b2r, W3)
    p3f, p3b = _sc_apply_feat(s3, packed3)
    g3, aw = _tc_attn_prep(p3f[:N], _decode_half(p3b), b3r, g1, g2, waT)

    pa = _sc_apply_scalar(aw.reshape(N), packed3)
    pre = pa[0, :N] + pa[1, :N] + ba[0]
    attn = jnp.tanh(pre)

    readout3, imp3 = _tc_readout(attn.reshape(B, 1, NPG), attn.reshape(N, 1),
                                 g1, g2, g3)
    return readout3.reshape(B, 6 * H), imp3.reshape(B * K)
